# Initial kernel scaffold; baseline (speedup 1.0000x reference)
import jax
import jax.numpy as jnp
from jax.experimental import pallas as pl


def kernel(node_embeds, edge_index, W_qkv, W_out):
    raise NotImplementedError("write your pallas kernel here")


# speedup vs baseline: 12.8403x; 12.8403x over previous
import functools

import jax
import jax.numpy as jnp
from jax import lax
from jax.experimental import pallas as pl
from jax.experimental.pallas import tpu as pltpu
from jax.experimental.pallas import tpu_sc as plsc

N_HEADS = 4
HEAD_DIM = 32
SCALE = HEAD_DIM ** (-0.5)
L = 16
NC, NS = 2, 16
NW = NC * NS
ACC_W = 144


def _qkv_body(x_ref, w_ref, q_ref, kv_ref):
    y = lax.dot_general(x_ref[...], w_ref[...], (((1,), (1,)), ((), ())),
                        preferred_element_type=jnp.float32)
    d = q_ref.shape[1]
    q_ref[...] = y[:, :d]
    kv_ref[...] = y[:, d:]


@functools.lru_cache(maxsize=None)
def _make_qkv(n, d, interpret):
    blk = 2000 if n % 2000 == 0 else n
    grid = n // blk
    return pl.pallas_call(
        _qkv_body,
        grid=(grid,),
        in_specs=[pl.BlockSpec((blk, d), lambda i: (i, 0)),
                  pl.BlockSpec((3 * d, d), lambda i: (0, 0))],
        out_specs=[pl.BlockSpec((blk, d), lambda i: (i, 0)),
                   pl.BlockSpec((blk, 2 * d), lambda i: (i, 0))],
        out_shape=[jax.ShapeDtypeStruct((n, d), jnp.float32),
                   jax.ShapeDtypeStruct((n, 2 * d), jnp.float32)],
        interpret=interpret,
    )


def _edge_body(q_hbm, kv_hbm, src_hbm, tgt_hbm, out_hbm,
               acc_sh, src_v, tgt_v, qrow_v, kvrow_v, wrow_v,
               sem_q, sem_kv, *, n_nodes, e_per_w, chunk):
    c = lax.axis_index("c")
    s = lax.axis_index("s")
    wid = s * NC + c
    n_chunks = e_per_w // chunk
    rows_per_tile = n_nodes // NS
    zrows = 25 if rows_per_tile % 25 == 0 else rows_per_tile

    def _zero_row(r, carry):
        for i in range(ACC_W // L):
            wrow_v[r, pl.ds(i * L, L)] = jnp.zeros((L,), jnp.float32)
        return carry
    lax.fori_loop(0, min(zrows, wrow_v.shape[0]), _zero_row, 0)
    for t in range(rows_per_tile // zrows):
        pltpu.sync_copy(wrow_v.at[pl.ds(0, zrows)],
                        acc_sh.at[pl.ds(s * rows_per_tile + t * zrows, zrows)])
    plsc.subcore_barrier()

    iota = lax.iota(jnp.int32, L)
    base_e = wid * e_per_w

    def _chunk(ci, carry):
        off = base_e + ci * chunk
        pltpu.sync_copy(src_hbm.at[pl.ds(off, chunk)], src_v)
        pltpu.sync_copy(tgt_hbm.at[pl.ds(off, chunk)], tgt_v)
        cq = pltpu.async_copy(q_hbm.at[src_v], qrow_v, sem_q)
        ckv = pltpu.async_copy(kv_hbm.at[tgt_v], kvrow_v, sem_kv)
        cq.wait()
        ckv.wait()

        def _edge(e, ecarry):
            att = jnp.zeros((L,), jnp.float32)
            for h in range(N_HEADS):
                pa = qrow_v[e, pl.ds(2 * h * L, L)] * kvrow_v[e, pl.ds(2 * h * L, L)]
                pb = qrow_v[e, pl.ds((2 * h + 1) * L, L)] * kvrow_v[e, pl.ds((2 * h + 1) * L, L)]
                sh = jnp.sum(pa + pb) * SCALE
                sh = jnp.clip(sh, -10.0, 10.0)
                ev = jnp.exp(jnp.full((L,), sh, jnp.float32))
                wrow_v[e, pl.ds(2 * h * L, L)] = kvrow_v[e, pl.ds((8 + 2 * h) * L, L)] * ev
                wrow_v[e, pl.ds((2 * h + 1) * L, L)] = kvrow_v[e, pl.ds((9 + 2 * h) * L, L)] * ev
                att = jnp.where(iota == h, ev, att)
            wrow_v[e, pl.ds(8 * L, L)] = att
            return ecarry
        lax.fori_loop(0, chunk, _edge, 0)

        pltpu.sync_copy(wrow_v, acc_sh.at[src_v], add=True)
        return carry
    lax.fori_loop(0, n_chunks, _chunk, 0)

    plsc.subcore_barrier()
    r0 = s * rows_per_tile
    pltpu.sync_copy(acc_sh.at[pl.ds(r0, rows_per_tile)],
                    out_hbm.at[c, pl.ds(r0, rows_per_tile)])


@functools.lru_cache(maxsize=None)
def _make_edge(n_nodes, n_edges, d, interpret):
    e_per_w = n_edges // NW
    chunk = 40 if e_per_w % 40 == 0 else e_per_w
    mesh = plsc.VectorSubcoreMesh(core_axis_name="c", subcore_axis_name="s",
                                  num_cores=NC, num_subcores=NS)
    body = functools.partial(_edge_body, n_nodes=n_nodes, e_per_w=e_per_w,
                             chunk=chunk)
    return pl.kernel(
        body,
        out_type=jax.ShapeDtypeStruct((NC, n_nodes, ACC_W), jnp.float32),
        mesh=mesh,
        scratch_types=[
            pltpu.VMEM_SHARED((n_nodes, ACC_W), jnp.float32),
            pltpu.VMEM((chunk,), jnp.int32),
            pltpu.VMEM((chunk,), jnp.int32),
            pltpu.VMEM((chunk, d), jnp.float32),
            pltpu.VMEM((chunk, 2 * d), jnp.float32),
            pltpu.VMEM((chunk, ACC_W), jnp.float32),
            pltpu.SemaphoreType.DMA,
            pltpu.SemaphoreType.DMA,
        ],
        compiler_params=pltpu.CompilerParams(use_tc_tiling_on_sc=False,
                                             needs_layout_passes=False),
        interpret=interpret,
    )


def _out_body(a0_ref, a1_ref, w_ref, o_ref):
    a = a0_ref[...] + a1_ref[...]
    d = o_ref.shape[1]
    att = a[:, d:d + N_HEADS]
    recip = 1.0 / (att + 1e-8)
    row = lax.broadcasted_iota(jnp.int32, (N_HEADS, d), 0)
    col = lax.broadcasted_iota(jnp.int32, (N_HEADS, d), 1)
    expand = (col // HEAD_DIM == row).astype(jnp.float32)
    scale = lax.dot_general(recip, expand, (((1,), (0,)), ((), ())),
                            preferred_element_type=jnp.float32)
    o = a[:, :d] * scale
    o_ref[...] = lax.dot_general(o, w_ref[...], (((1,), (1,)), ((), ())),
                                 preferred_element_type=jnp.float32)


@functools.lru_cache(maxsize=None)
def _make_out(n, d, interpret):
    blk = 2000 if n % 2000 == 0 else n
    grid = n // blk
    return pl.pallas_call(
        _out_body,
        grid=(grid,),
        in_specs=[pl.BlockSpec((blk, ACC_W), lambda i: (i, 0)),
                  pl.BlockSpec((blk, ACC_W), lambda i: (i, 0)),
                  pl.BlockSpec((d, d), lambda i: (0, 0))],
        out_specs=pl.BlockSpec((blk, d), lambda i: (i, 0)),
        out_shape=jax.ShapeDtypeStruct((n, d), jnp.float32),
        interpret=interpret,
    )


def kernel(node_embeds, edge_index, W_qkv, W_out, *, interpret=False):
    n, d = node_embeds.shape
    n_edges = edge_index.shape[1]
    q, kv = _make_qkv(n, d, interpret)(node_embeds, W_qkv)
    src = edge_index[0]
    tgt = edge_index[1]
    acc = _make_edge(n, n_edges, d, interpret)(q, kv, src, tgt)
    return _make_out(n, d, interpret)(acc[0], acc[1], W_out)


# speedup vs baseline: 14.7545x; 1.1491x over previous
import functools

import jax
import jax.numpy as jnp
from jax import lax
from jax.experimental import pallas as pl
from jax.experimental.pallas import tpu as pltpu
from jax.experimental.pallas import tpu_sc as plsc

N_HEADS = 4
HEAD_DIM = 32
SCALE = HEAD_DIM ** (-0.5)
L = 16
NC, NS = 2, 16
NW = NC * NS
ACC_W = 144


def _qkv_body(x_ref, w_ref, q_ref, kv_ref):
    y = lax.dot_general(x_ref[...], w_ref[...], (((1,), (1,)), ((), ())),
                        preferred_element_type=jnp.float32)
    d = q_ref.shape[1]
    q_ref[...] = y[:, :d]
    kv_ref[...] = y[:, d:]


@functools.lru_cache(maxsize=None)
def _make_qkv(n, d, interpret):
    blk = 2000 if n % 2000 == 0 else n
    grid = n // blk
    return pl.pallas_call(
        _qkv_body,
        grid=(grid,),
        in_specs=[pl.BlockSpec((blk, d), lambda i: (i, 0)),
                  pl.BlockSpec((3 * d, d), lambda i: (0, 0))],
        out_specs=[pl.BlockSpec((blk, d), lambda i: (i, 0)),
                   pl.BlockSpec((blk, 2 * d), lambda i: (i, 0))],
        out_shape=[jax.ShapeDtypeStruct((n, d), jnp.float32),
                   jax.ShapeDtypeStruct((n, 2 * d), jnp.float32)],
        interpret=interpret,
    )


def _edge_body(q_hbm, kv_hbm, src_hbm, tgt_hbm, out_hbm,
               acc_sh, src0, tgt0, src1, tgt1, q0, q1, kv0, kv1, wrow_v,
               semq0, semkv0, semq1, semkv1, *, n_nodes, e_per_w, chunk):
    c = lax.axis_index("c")
    s = lax.axis_index("s")
    wid = s * NC + c
    n_chunks = e_per_w // chunk
    n_pairs = n_chunks // 2
    rows_per_tile = n_nodes // NS
    zrows = 25 if rows_per_tile % 25 == 0 else rows_per_tile

    def _zero_row(r, carry):
        for i in range(ACC_W // L):
            wrow_v[r, pl.ds(i * L, L)] = jnp.zeros((L,), jnp.float32)
        return carry
    lax.fori_loop(0, min(zrows, wrow_v.shape[0]), _zero_row, 0)
    for t in range(rows_per_tile // zrows):
        pltpu.sync_copy(wrow_v.at[pl.ds(0, zrows)],
                        acc_sh.at[pl.ds(s * rows_per_tile + t * zrows, zrows)])
    plsc.subcore_barrier()

    iota = lax.iota(jnp.int32, L)
    base_e = wid * e_per_w

    def _fetch(off, src_v, tgt_v, qb, kvb, sq, skv):
        pltpu.sync_copy(src_hbm.at[pl.ds(off, chunk)], src_v)
        pltpu.sync_copy(tgt_hbm.at[pl.ds(off, chunk)], tgt_v)
        pltpu.async_copy(q_hbm.at[src_v], qb, sq)
        pltpu.async_copy(kv_hbm.at[tgt_v], kvb, skv)

    def _compute_scatter(src_v, tgt_v, qb, kvb, sq, skv):
        pltpu.make_async_copy(q_hbm.at[src_v], qb, sq).wait()
        pltpu.make_async_copy(kv_hbm.at[tgt_v], kvb, skv).wait()

        def _edge(e, ecarry):
            att = jnp.zeros((L,), jnp.float32)
            for h in range(N_HEADS):
                pa = qb[e, pl.ds(2 * h * L, L)] * kvb[e, pl.ds(2 * h * L, L)]
                pb = qb[e, pl.ds((2 * h + 1) * L, L)] * kvb[e, pl.ds((2 * h + 1) * L, L)]
                sh = jnp.sum(pa + pb) * SCALE
                sh = jnp.clip(sh, -10.0, 10.0)
                ev = jnp.exp(jnp.full((L,), sh, jnp.float32))
                wrow_v[e, pl.ds(2 * h * L, L)] = kvb[e, pl.ds((8 + 2 * h) * L, L)] * ev
                wrow_v[e, pl.ds((2 * h + 1) * L, L)] = kvb[e, pl.ds((9 + 2 * h) * L, L)] * ev
                att = jnp.where(iota == h, ev, att)
            wrow_v[e, pl.ds(8 * L, L)] = att
            return ecarry
        lax.fori_loop(0, chunk, _edge, 0, unroll=2)

        pltpu.sync_copy(wrow_v, acc_sh.at[src_v], add=True)

    _fetch(base_e, src0, tgt0, q0, kv0, semq0, semkv0)

    def _pair(t, carry):
        _fetch(base_e + (2 * t + 1) * chunk, src1, tgt1, q1, kv1, semq1, semkv1)
        _compute_scatter(src0, tgt0, q0, kv0, semq0, semkv0)

        @pl.when(t < n_pairs - 1)
        def _():
            _fetch(base_e + (2 * t + 2) * chunk, src0, tgt0, q0, kv0,
                   semq0, semkv0)
        _compute_scatter(src1, tgt1, q1, kv1, semq1, semkv1)
        return carry
    lax.fori_loop(0, n_pairs, _pair, 0)

    plsc.subcore_barrier()
    r0 = s * rows_per_tile
    pltpu.sync_copy(acc_sh.at[pl.ds(r0, rows_per_tile)],
                    out_hbm.at[c, pl.ds(r0, rows_per_tile)])


@functools.lru_cache(maxsize=None)
def _make_edge(n_nodes, n_edges, d, interpret):
    e_per_w = n_edges // NW
    chunk = 40 if e_per_w % 80 == 0 else e_per_w // 2
    mesh = plsc.VectorSubcoreMesh(core_axis_name="c", subcore_axis_name="s",
                                  num_cores=NC, num_subcores=NS)
    body = functools.partial(_edge_body, n_nodes=n_nodes, e_per_w=e_per_w,
                             chunk=chunk)
    return pl.kernel(
        body,
        out_type=jax.ShapeDtypeStruct((NC, n_nodes, ACC_W), jnp.float32),
        mesh=mesh,
        scratch_types=[
            pltpu.VMEM_SHARED((n_nodes, ACC_W), jnp.float32),
            pltpu.VMEM((chunk,), jnp.int32),
            pltpu.VMEM((chunk,), jnp.int32),
            pltpu.VMEM((chunk,), jnp.int32),
            pltpu.VMEM((chunk,), jnp.int32),
            pltpu.VMEM((chunk, d), jnp.float32),
            pltpu.VMEM((chunk, d), jnp.float32),
            pltpu.VMEM((chunk, 2 * d), jnp.float32),
            pltpu.VMEM((chunk, 2 * d), jnp.float32),
            pltpu.VMEM((chunk, ACC_W), jnp.float32),
            pltpu.SemaphoreType.DMA,
            pltpu.SemaphoreType.DMA,
            pltpu.SemaphoreType.DMA,
            pltpu.SemaphoreType.DMA,
        ],
        compiler_params=pltpu.CompilerParams(use_tc_tiling_on_sc=False,
                                             needs_layout_passes=False),
        interpret=interpret,
    )


def _out_body(a0_ref, a1_ref, w_ref, o_ref):
    a = a0_ref[...] + a1_ref[...]
    d = o_ref.shape[1]
    att = a[:, d:d + N_HEADS]
    recip = 1.0 / (att + 1e-8)
    row = lax.broadcasted_iota(jnp.int32, (N_HEADS, d), 0)
    col = lax.broadcasted_iota(jnp.int32, (N_HEADS, d), 1)
    expand = (col // HEAD_DIM == row).astype(jnp.float32)
    scale = lax.dot_general(recip, expand, (((1,), (0,)), ((), ())),
                            preferred_element_type=jnp.float32)
    o = a[:, :d] * scale
    o_ref[...] = lax.dot_general(o, w_ref[...], (((1,), (1,)), ((), ())),
                                 preferred_element_type=jnp.float32)


@functools.lru_cache(maxsize=None)
def _make_out(n, d, interpret):
    blk = 2000 if n % 2000 == 0 else n
    grid = n // blk
    return pl.pallas_call(
        _out_body,
        grid=(grid,),
        in_specs=[pl.BlockSpec((blk, ACC_W), lambda i: (i, 0)),
                  pl.BlockSpec((blk, ACC_W), lambda i: (i, 0)),
                  pl.BlockSpec((d, d), lambda i: (0, 0))],
        out_specs=pl.BlockSpec((blk, d), lambda i: (i, 0)),
        out_shape=jax.ShapeDtypeStruct((n, d), jnp.float32),
        interpret=interpret,
    )


def kernel(node_embeds, edge_index, W_qkv, W_out, *, interpret=False):
    n, d = node_embeds.shape
    n_edges = edge_index.shape[1]
    q, kv = _make_qkv(n, d, interpret)(node_embeds, W_qkv)
    src = edge_index[0]
    tgt = edge_index[1]
    acc = _make_edge(n, n_edges, d, interpret)(q, kv, src, tgt)
    return _make_out(n, d, interpret)(acc[0], acc[1], W_out)


# speedup vs baseline: 14.7639x; 1.0006x over previous
import functools

import jax
import jax.numpy as jnp
from jax import lax
from jax.experimental import pallas as pl
from jax.experimental.pallas import tpu as pltpu
from jax.experimental.pallas import tpu_sc as plsc

N_HEADS = 4
HEAD_DIM = 32
SCALE = HEAD_DIM ** (-0.5)
L = 16
NC, NS = 2, 16
NW = NC * NS
ACC_W = 144


def _qkv_body(x_ref, w_ref, q_ref, kv_ref):
    y = lax.dot_general(x_ref[...], w_ref[...], (((1,), (1,)), ((), ())),
                        preferred_element_type=jnp.float32)
    d = q_ref.shape[1]
    q_ref[...] = y[:, :d]
    kv_ref[...] = y[:, d:]


@functools.lru_cache(maxsize=None)
def _make_qkv(n, d, interpret):
    blk = 2000 if n % 2000 == 0 else n
    grid = n // blk
    return pl.pallas_call(
        _qkv_body,
        grid=(grid,),
        in_specs=[pl.BlockSpec((blk, d), lambda i: (i, 0)),
                  pl.BlockSpec((3 * d, d), lambda i: (0, 0))],
        out_specs=[pl.BlockSpec((blk, d), lambda i: (i, 0)),
                   pl.BlockSpec((blk, 2 * d), lambda i: (i, 0))],
        out_shape=[jax.ShapeDtypeStruct((n, d), jnp.float32),
                   jax.ShapeDtypeStruct((n, 2 * d), jnp.float32)],
        interpret=interpret,
    )


def _edge_body(q_hbm, kv_hbm, src_hbm, tgt_hbm, out_hbm,
               acc_sh, src0, tgt0, src1, tgt1, q0, q1, kv0, kv1, wrow_v,
               semq0, semkv0, semq1, semkv1, *, n_nodes, e_per_w, chunk):
    c = lax.axis_index("c")
    s = lax.axis_index("s")
    wid = s * NC + c
    n_chunks = e_per_w // chunk
    n_pairs = n_chunks // 2
    rows_per_tile = n_nodes // NS
    zrows = 25 if rows_per_tile % 25 == 0 else rows_per_tile

    def _zero_row(r, carry):
        for i in range(ACC_W // L):
            wrow_v[r, pl.ds(i * L, L)] = jnp.zeros((L,), jnp.float32)
        return carry
    lax.fori_loop(0, min(zrows, wrow_v.shape[0]), _zero_row, 0)
    for t in range(rows_per_tile // zrows):
        pltpu.sync_copy(wrow_v.at[pl.ds(0, zrows)],
                        acc_sh.at[pl.ds(s * rows_per_tile + t * zrows, zrows)])
    plsc.subcore_barrier()

    iota = lax.iota(jnp.int32, L)
    base_e = wid * e_per_w

    def _fetch(off, src_v, tgt_v, qb, kvb, sq, skv):
        pltpu.sync_copy(src_hbm.at[pl.ds(off, chunk)], src_v)
        pltpu.sync_copy(tgt_hbm.at[pl.ds(off, chunk)], tgt_v)
        pltpu.async_copy(q_hbm.at[src_v], qb, sq)
        pltpu.async_copy(kv_hbm.at[tgt_v], kvb, skv)

    def _compute_scatter(src_v, tgt_v, qb, kvb, sq, skv):
        pltpu.make_async_copy(q_hbm.at[src_v], qb, sq).wait()
        pltpu.make_async_copy(kv_hbm.at[tgt_v], kvb, skv).wait()

        def _edge(e, ecarry):
            att = jnp.zeros((L,), jnp.float32)
            for h in range(N_HEADS):
                pa = qb[e, pl.ds(2 * h * L, L)] * kvb[e, pl.ds(2 * h * L, L)]
                pb = qb[e, pl.ds((2 * h + 1) * L, L)] * kvb[e, pl.ds((2 * h + 1) * L, L)]
                sh = jnp.sum(pa + pb) * SCALE
                sh = jnp.clip(sh, -10.0, 10.0)
                ev = jnp.exp(jnp.full((L,), sh, jnp.float32))
                wrow_v[e, pl.ds(2 * h * L, L)] = kvb[e, pl.ds((8 + 2 * h) * L, L)] * ev
                wrow_v[e, pl.ds((2 * h + 1) * L, L)] = kvb[e, pl.ds((9 + 2 * h) * L, L)] * ev
                att = jnp.where(iota == h, ev, att)
            wrow_v[e, pl.ds(8 * L, L)] = att
            return ecarry
        lax.fori_loop(0, chunk, _edge, 0, unroll=4)

        pltpu.sync_copy(wrow_v, acc_sh.at[src_v], add=True)

    _fetch(base_e, src0, tgt0, q0, kv0, semq0, semkv0)

    def _pair(t, carry):
        _fetch(base_e + (2 * t + 1) * chunk, src1, tgt1, q1, kv1, semq1, semkv1)
        _compute_scatter(src0, tgt0, q0, kv0, semq0, semkv0)

        @pl.when(t < n_pairs - 1)
        def _():
            _fetch(base_e + (2 * t + 2) * chunk, src0, tgt0, q0, kv0,
                   semq0, semkv0)
        _compute_scatter(src1, tgt1, q1, kv1, semq1, semkv1)
        return carry
    lax.fori_loop(0, n_pairs, _pair, 0)

    plsc.subcore_barrier()
    r0 = s * rows_per_tile
    pltpu.sync_copy(acc_sh.at[pl.ds(r0, rows_per_tile)],
                    out_hbm.at[c, pl.ds(r0, rows_per_tile)])


@functools.lru_cache(maxsize=None)
def _make_edge(n_nodes, n_edges, d, interpret):
    e_per_w = n_edges // NW
    chunk = 40 if e_per_w % 80 == 0 else e_per_w // 2
    mesh = plsc.VectorSubcoreMesh(core_axis_name="c", subcore_axis_name="s",
                                  num_cores=NC, num_subcores=NS)
    body = functools.partial(_edge_body, n_nodes=n_nodes, e_per_w=e_per_w,
                             chunk=chunk)
    return pl.kernel(
        body,
        out_type=jax.ShapeDtypeStruct((NC, n_nodes, ACC_W), jnp.float32),
        mesh=mesh,
        scratch_types=[
            pltpu.VMEM_SHARED((n_nodes, ACC_W), jnp.float32),
            pltpu.VMEM((chunk,), jnp.int32),
            pltpu.VMEM((chunk,), jnp.int32),
            pltpu.VMEM((chunk,), jnp.int32),
            pltpu.VMEM((chunk,), jnp.int32),
            pltpu.VMEM((chunk, d), jnp.float32),
            pltpu.VMEM((chunk, d), jnp.float32),
            pltpu.VMEM((chunk, 2 * d), jnp.float32),
            pltpu.VMEM((chunk, 2 * d), jnp.float32),
            pltpu.VMEM((chunk, ACC_W), jnp.float32),
            pltpu.SemaphoreType.DMA,
            pltpu.SemaphoreType.DMA,
            pltpu.SemaphoreType.DMA,
            pltpu.SemaphoreType.DMA,
        ],
        compiler_params=pltpu.CompilerParams(use_tc_tiling_on_sc=False,
                                             needs_layout_passes=False),
        interpret=interpret,
    )


def _out_body(a0_ref, a1_ref, w_ref, o_ref):
    a = a0_ref[...] + a1_ref[...]
    d = o_ref.shape[1]
    att = a[:, d:d + N_HEADS]
    recip = 1.0 / (att + 1e-8)
    row = lax.broadcasted_iota(jnp.int32, (N_HEADS, d), 0)
    col = lax.broadcasted_iota(jnp.int32, (N_HEADS, d), 1)
    expand = (col // HEAD_DIM == row).astype(jnp.float32)
    scale = lax.dot_general(recip, expand, (((1,), (0,)), ((), ())),
                            preferred_element_type=jnp.float32)
    o = a[:, :d] * scale
    o_ref[...] = lax.dot_general(o, w_ref[...], (((1,), (1,)), ((), ())),
                                 preferred_element_type=jnp.float32)


@functools.lru_cache(maxsize=None)
def _make_out(n, d, interpret):
    blk = 2000 if n % 2000 == 0 else n
    grid = n // blk
    return pl.pallas_call(
        _out_body,
        grid=(grid,),
        in_specs=[pl.BlockSpec((blk, ACC_W), lambda i: (i, 0)),
                  pl.BlockSpec((blk, ACC_W), lambda i: (i, 0)),
                  pl.BlockSpec((d, d), lambda i: (0, 0))],
        out_specs=pl.BlockSpec((blk, d), lambda i: (i, 0)),
        out_shape=jax.ShapeDtypeStruct((n, d), jnp.float32),
        interpret=interpret,
    )


def kernel(node_embeds, edge_index, W_qkv, W_out, *, interpret=False):
    n, d = node_embeds.shape
    n_edges = edge_index.shape[1]
    q, kv = _make_qkv(n, d, interpret)(node_embeds, W_qkv)
    src = edge_index[0]
    tgt = edge_index[1]
    acc = _make_edge(n, n_edges, d, interpret)(q, kv, src, tgt)
    return _make_out(n, d, interpret)(acc[0], acc[1], W_out)
